# baseline (speedup 1.0000x reference)
import functools

import jax
import jax.numpy as jnp
from jax import lax
from jax.experimental import pallas as pl
from jax.experimental.pallas import tpu as pltpu
from jax.experimental.pallas import tpu_sc as plsc

H = 12
Q = 2048
K = 2048
DH = 64
NE = 22
NW = 32
QPW = Q // NW
TROW = QPW * NE
L = 16
GROUPS = K // L


def _scores_body(q_ref, w_ref, o_ref):
    w = w_ref[0]
    nmask = lax.broadcasted_iota(jnp.int32, (DH, NE), 1) != 0
    w = jnp.where(nmask, w, 0.0)
    o_ref[0] = jnp.dot(q_ref[0], w, preferred_element_type=jnp.float32)


def _compute_scores(q, w):
    return pl.pallas_call(
        _scores_body,
        grid=(H,),
        in_specs=[
            pl.BlockSpec((1, Q, DH), lambda h: (h, 0, 0)),
            pl.BlockSpec((1, DH, NE), lambda h: (h, 0, 0)),
        ],
        out_specs=pl.BlockSpec((1, Q, NE), lambda h: (h, 0, 0)),
        out_shape=jax.ShapeDtypeStruct((H, Q, NE), jnp.float32),
    )(q, w)


QB = 2
NB = QPW // QB


def _gather_body(scores_hbm, rp_hbm, out_hbm, table_v, rp_v, out_v,
                 rp_sem, out_sem0, out_sem1):
    c = lax.axis_index("c")
    s = lax.axis_index("s")
    wid = s * 2 + c
    q0 = wid * QPW
    for h in range(H):
        pltpu.sync_copy(scores_hbm.at[h, pl.ds(q0, QPW), :], table_v.at[h])
    pltpu.async_copy(rp_hbm.at[pl.ds(q0, QB)], rp_v.at[0], rp_sem)
    out_sems = (out_sem0, out_sem1)

    def pair_body(qp, carry):
        for b in range(2):
            ql = (2 * qp + b) * QB
            q = q0 + ql
            band = q // 8
            sub = q % 8
            pltpu.make_async_copy(
                rp_hbm.at[pl.ds(q, QB)], rp_v.at[b], rp_sem).wait()

            @pl.when(ql + QB < QPW)
            def _():
                pltpu.async_copy(
                    rp_hbm.at[pl.ds(q + QB, QB)], rp_v.at[1 - b], rp_sem)

            @pl.when(qp > 0)
            def _():
                for qq in range(QB):
                    pltpu.make_async_copy(
                        out_v.at[b, :, qq],
                        out_hbm.at[0, :, band, :, sub + qq, :],
                        out_sems[b]).wait()

            qsp = [jnp.full((L,), ql + qq, jnp.int32) for qq in range(QB)]

            @plsc.parallel_loop(0, 16)
            def gat(tc, b=b, qsp=qsp):
                for lg in range(8):
                    for qq in range(QB):
                        idx = rp_v[b, qq, pl.ds(tc * 128 + lg * L, L)]
                        vals = [plsc.load_gather(table_v.at[h],
                                                 [qsp[qq], idx])
                                for h in range(H)]
                        for h in range(H):
                            out_v[b, h, qq, tc, pl.ds(lg * L, L)] = vals[h]
            for qq in range(QB):
                pltpu.async_copy(
                    out_v.at[b, :, qq],
                    out_hbm.at[0, :, band, :, sub + qq, :], out_sems[b])
        return carry

    lax.fori_loop(0, NB // 2, pair_body, 0)
    qe = q0 + QPW - 2 * QB
    for b in range(2):
        for qq in range(QB):
            pltpu.make_async_copy(
                out_v.at[b, :, qq],
                out_hbm.at[0, :, (qe + b * QB) // 8, :,
                           (qe + b * QB) % 8 + qq, :],
                out_sems[b]).wait()


def _gather(scores, rp):
    mesh = plsc.VectorSubcoreMesh(core_axis_name="c", subcore_axis_name="s")
    f = pl.kernel(
        _gather_body,
        out_type=jax.ShapeDtypeStruct((1, H, 256, 16, 8, 128), jnp.float32),
        mesh=mesh,
        compiler_params=pltpu.CompilerParams(
            use_tc_tiling_on_sc=False, needs_layout_passes=False),
        scratch_types=[
            pltpu.VMEM((H, QPW, NE), jnp.float32),
            pltpu.VMEM((2, QB, K), jnp.int32),
            pltpu.VMEM((2, H, QB, 16, 128), jnp.float32),
            pltpu.SemaphoreType.DMA,
            pltpu.SemaphoreType.DMA,
            pltpu.SemaphoreType.DMA,
        ],
    )
    return f(scores, rp)


@jax.jit
def kernel(query, relpos, emb_weight):
    q = query[0]
    rp = relpos[0, :, :, 0]
    w = emb_weight.reshape(NE, H, DH).transpose(1, 2, 0)
    scores = _compute_scores(q, w)
    out = _gather(scores, rp)
    out = out.transpose(0, 1, 2, 4, 3, 5)
    return out.reshape(1, H, Q, K)


# speedup vs baseline: 1.0806x; 1.0806x over previous
import functools

import jax
import jax.numpy as jnp
from jax import lax
from jax.experimental import pallas as pl
from jax.experimental.pallas import tpu as pltpu
from jax.experimental.pallas import tpu_sc as plsc

H = 12
Q = 2048
K = 2048
DH = 64
NE = 22
NW = 32
QPW = Q // NW
TROW = QPW * NE
L = 16
GROUPS = K // L


def _scores_body(q_ref, w_ref, o_ref):
    w = w_ref[0]
    nmask = lax.broadcasted_iota(jnp.int32, (DH, NE), 1) != 0
    w = jnp.where(nmask, w, 0.0)
    o_ref[0] = jnp.dot(q_ref[0], w, preferred_element_type=jnp.float32)


def _compute_scores(q, w):
    return pl.pallas_call(
        _scores_body,
        grid=(H,),
        in_specs=[
            pl.BlockSpec((1, Q, DH), lambda h: (h, 0, 0)),
            pl.BlockSpec((1, DH, NE), lambda h: (h, 0, 0)),
        ],
        out_specs=pl.BlockSpec((1, Q, NE), lambda h: (h, 0, 0)),
        out_shape=jax.ShapeDtypeStruct((H, Q, NE), jnp.float32),
    )(q, w)


QB = 2
NB = QPW // QB


def _gather_body(scores_hbm, rp_hbm, out_hbm, table_v, rp_v, out_v,
                 rp_sem, out_sem0, out_sem1):
    c = lax.axis_index("c")
    s = lax.axis_index("s")
    wid = s * 2 + c
    q0 = wid * QPW
    for h in range(H):
        pltpu.sync_copy(scores_hbm.at[h, wid], table_v.at[h])
    pltpu.async_copy(rp_hbm.at[pl.ds(q0, QB)], rp_v.at[0], rp_sem)
    out_sems = (out_sem0, out_sem1)

    def pair_body(qp, carry):
        for b in range(2):
            ql = (2 * qp + b) * QB
            q = q0 + ql
            band = q // 8
            sub = q % 8
            pltpu.make_async_copy(
                rp_hbm.at[pl.ds(q, QB)], rp_v.at[b], rp_sem).wait()

            @pl.when(ql + QB < QPW)
            def _():
                pltpu.async_copy(
                    rp_hbm.at[pl.ds(q + QB, QB)], rp_v.at[1 - b], rp_sem)

            @pl.when(qp > 0)
            def _():
                for qq in range(QB):
                    pltpu.make_async_copy(
                        out_v.at[b, :, qq],
                        out_hbm.at[0, :, band, :, sub + qq, :],
                        out_sems[b]).wait()

            @plsc.parallel_loop(0, 16)
            def gat(tc, b=b, ql=ql):
                for lg in range(8):
                    for qq in range(QB):
                        idx = (rp_v[b, qq, pl.ds(tc * 128 + lg * L, L)]
                               + (ql + qq) * NE)
                        vals = [plsc.load_gather(table_v.at[h], [idx])
                                for h in range(H)]
                        for h in range(H):
                            out_v[b, h, qq, tc, pl.ds(lg * L, L)] = vals[h]
            for qq in range(QB):
                pltpu.async_copy(
                    out_v.at[b, :, qq],
                    out_hbm.at[0, :, band, :, sub + qq, :], out_sems[b])
        return carry

    lax.fori_loop(0, NB // 2, pair_body, 0)
    qe = q0 + QPW - 2 * QB
    for b in range(2):
        for qq in range(QB):
            pltpu.make_async_copy(
                out_v.at[b, :, qq],
                out_hbm.at[0, :, (qe + b * QB) // 8, :,
                           (qe + b * QB) % 8 + qq, :],
                out_sems[b]).wait()


def _gather(scores, rp):
    mesh = plsc.VectorSubcoreMesh(core_axis_name="c", subcore_axis_name="s")
    f = pl.kernel(
        _gather_body,
        out_type=jax.ShapeDtypeStruct((1, H, 256, 16, 8, 128), jnp.float32),
        mesh=mesh,
        compiler_params=pltpu.CompilerParams(
            use_tc_tiling_on_sc=False, needs_layout_passes=False),
        scratch_types=[
            pltpu.VMEM((H, TROW), jnp.float32),
            pltpu.VMEM((2, QB, K), jnp.int32),
            pltpu.VMEM((2, H, QB, 16, 128), jnp.float32),
            pltpu.SemaphoreType.DMA,
            pltpu.SemaphoreType.DMA,
            pltpu.SemaphoreType.DMA,
        ],
    )
    return f(scores, rp)


@jax.jit
def kernel(query, relpos, emb_weight):
    q = query[0]
    rp = relpos[0, :, :, 0]
    w = emb_weight.reshape(NE, H, DH).transpose(1, 2, 0)
    scores = _compute_scores(q, w)
    scores = scores.reshape(H, NW, TROW)
    out = _gather(scores, rp)
    out = out.transpose(0, 1, 2, 4, 3, 5)
    return out.reshape(1, H, Q, K)
